# baseline (speedup 1.0000x reference)
import jax
import jax.numpy as jnp
from jax.experimental import pallas as pl
from jax.experimental.pallas import tpu as pltpu

B = 128
K = 8
LOCAL_E = 8
H = 2048
I = 1024


def _ffn_body(ids_ref, scl_ref, act_ref, x_ref, smooth_ref,
              w1_ref, s1_ref, w2_ref, s2_ref,
              out_ref):
    e = pl.program_id(0)

    m = (ids_ref[...] == e).astype(jnp.float32)
    w_col = jnp.sum(m * scl_ref[...], axis=1, keepdims=True) * act_ref[...]

    xs = x_ref[...] * smooth_ref[0]
    h = jnp.dot(xs, w1_ref[0], preferred_element_type=jnp.float32) * s1_ref[0]
    g = h[:, :I]
    u = h[:, I:]
    a = (g * jax.nn.sigmoid(g)) * u
    part = jnp.dot(a, w2_ref[0], preferred_element_type=jnp.float32)

    @pl.when(e == 0)
    def _():
        out_ref[...] = jnp.zeros_like(out_ref)

    out_ref[...] += part * s2_ref[0] * w_col


def kernel(x, expert_ids, smooth_scales, expert_scales, x_active_mask,
           gmm1_weight, gmm1_weight_scale, gmm2_weight, gmm2_weight_scale):
    act_col = x_active_mask.astype(jnp.float32).reshape(B, 1)
    smooth3 = smooth_scales.reshape(LOCAL_E, 1, H)
    s1_3 = gmm1_weight_scale.reshape(LOCAL_E, 1, 2 * I)
    s2_3 = gmm2_weight_scale.reshape(LOCAL_E, 1, H)

    grid = (LOCAL_E,)
    out = pl.pallas_call(
        _ffn_body,
        grid=grid,
        in_specs=[
            pl.BlockSpec((B, K), lambda e: (0, 0)),
            pl.BlockSpec((B, K), lambda e: (0, 0)),
            pl.BlockSpec((B, 1), lambda e: (0, 0)),
            pl.BlockSpec((B, H), lambda e: (0, 0)),
            pl.BlockSpec((1, 1, H), lambda e: (e, 0, 0)),
            pl.BlockSpec((1, H, 2 * I), lambda e: (e, 0, 0)),
            pl.BlockSpec((1, 1, 2 * I), lambda e: (e, 0, 0)),
            pl.BlockSpec((1, I, H), lambda e: (e, 0, 0)),
            pl.BlockSpec((1, 1, H), lambda e: (e, 0, 0)),
        ],
        out_specs=pl.BlockSpec((B, H), lambda e: (0, 0)),
        out_shape=jax.ShapeDtypeStruct((B, H), jnp.float32),
        compiler_params=pltpu.CompilerParams(
            dimension_semantics=("arbitrary",),
        ),
    )(expert_ids, expert_scales, act_col, x, smooth3,
      gmm1_weight, s1_3, gmm2_weight, s2_3)
    return out


# speedup vs baseline: 1.0385x; 1.0385x over previous
import jax
import jax.numpy as jnp
from jax.experimental import pallas as pl
from jax.experimental.pallas import tpu as pltpu

B = 128
K = 8
LOCAL_E = 8
H = 2048
I = 1024
IT = 512
NI = I // IT


def _ffn_body(ids_ref, scl_ref, act_ref, x_ref, smooth_ref,
              w1g_ref, w1u_ref, s1g_ref, s1u_ref, w2_ref, s2_ref,
              out_ref):
    e = pl.program_id(0)
    i = pl.program_id(1)

    m = (ids_ref[...] == e).astype(jnp.float32)
    w_col = jnp.sum(m * scl_ref[...], axis=1, keepdims=True) * act_ref[...]

    xs = x_ref[...] * smooth_ref[0]
    g = jnp.dot(xs, w1g_ref[0], preferred_element_type=jnp.float32) * s1g_ref[0]
    u = jnp.dot(xs, w1u_ref[0], preferred_element_type=jnp.float32) * s1u_ref[0]
    a = (g * jax.nn.sigmoid(g)) * u
    part = jnp.dot(a, w2_ref[0], preferred_element_type=jnp.float32)

    @pl.when((e == 0) & (i == 0))
    def _():
        out_ref[...] = jnp.zeros_like(out_ref)

    out_ref[...] += part * s2_ref[0] * w_col


def kernel(x, expert_ids, smooth_scales, expert_scales, x_active_mask,
           gmm1_weight, gmm1_weight_scale, gmm2_weight, gmm2_weight_scale):
    act_col = x_active_mask.astype(jnp.float32).reshape(B, 1)
    smooth3 = smooth_scales.reshape(LOCAL_E, 1, H)
    s1_3 = gmm1_weight_scale.reshape(LOCAL_E, 1, 2 * I)
    s2_3 = gmm2_weight_scale.reshape(LOCAL_E, 1, H)

    grid = (LOCAL_E, NI)
    out = pl.pallas_call(
        _ffn_body,
        grid=grid,
        in_specs=[
            pl.BlockSpec((B, K), lambda e, i: (0, 0)),
            pl.BlockSpec((B, K), lambda e, i: (0, 0)),
            pl.BlockSpec((B, 1), lambda e, i: (0, 0)),
            pl.BlockSpec((B, H), lambda e, i: (0, 0)),
            pl.BlockSpec((1, 1, H), lambda e, i: (e, 0, 0)),
            pl.BlockSpec((1, H, IT), lambda e, i: (e, 0, i)),
            pl.BlockSpec((1, H, IT), lambda e, i: (e, 0, NI + i)),
            pl.BlockSpec((1, 1, IT), lambda e, i: (e, 0, i)),
            pl.BlockSpec((1, 1, IT), lambda e, i: (e, 0, NI + i)),
            pl.BlockSpec((1, IT, H), lambda e, i: (e, i, 0)),
            pl.BlockSpec((1, 1, H), lambda e, i: (e, 0, 0)),
        ],
        out_specs=pl.BlockSpec((B, H), lambda e, i: (0, 0)),
        out_shape=jax.ShapeDtypeStruct((B, H), jnp.float32),
        compiler_params=pltpu.CompilerParams(
            dimension_semantics=("arbitrary", "arbitrary"),
        ),
    )(expert_ids, expert_scales, act_col, x, smooth3,
      gmm1_weight, gmm1_weight, s1_3, s1_3,
      gmm2_weight, s2_3)
    return out


# speedup vs baseline: 1.0702x; 1.0305x over previous
import jax
import jax.numpy as jnp
from jax.experimental import pallas as pl
from jax.experimental.pallas import tpu as pltpu

B = 128
K = 8
LOCAL_E = 8
H = 2048
HH = H // 2
I = 1024
IT = 512
NI = I // IT


def _ffn_body(ids_ref, scl_ref, act_ref, x_ref, smooth_ref,
              w1g_t_ref, w1g_b_ref, w1u_t_ref, w1u_b_ref,
              s1g_ref, s1u_ref, w2_a_ref, w2_b_ref, s2_ref,
              out_ref):
    e = pl.program_id(0)
    i = pl.program_id(1)

    m = (ids_ref[...] == e).astype(jnp.float32)
    w_col = jnp.sum(m * scl_ref[...], axis=1, keepdims=True) * act_ref[...]

    xs = x_ref[...] * smooth_ref[0]
    xs_t = xs[:, :HH]
    xs_b = xs[:, HH:]
    g = (jnp.dot(xs_t, w1g_t_ref[0], preferred_element_type=jnp.float32)
         + jnp.dot(xs_b, w1g_b_ref[0], preferred_element_type=jnp.float32)) * s1g_ref[0]
    u = (jnp.dot(xs_t, w1u_t_ref[0], preferred_element_type=jnp.float32)
         + jnp.dot(xs_b, w1u_b_ref[0], preferred_element_type=jnp.float32)) * s1u_ref[0]
    a = (g * jax.nn.sigmoid(g)) * u
    part_a = jnp.dot(a, w2_a_ref[0], preferred_element_type=jnp.float32)
    part_b = jnp.dot(a, w2_b_ref[0], preferred_element_type=jnp.float32)

    @pl.when((e == 0) & (i == 0))
    def _():
        out_ref[...] = jnp.zeros_like(out_ref)

    s2 = s2_ref[0]
    out_ref[:, :HH] += part_a * s2[:, :HH] * w_col
    out_ref[:, HH:] += part_b * s2[:, HH:] * w_col


def kernel(x, expert_ids, smooth_scales, expert_scales, x_active_mask,
           gmm1_weight, gmm1_weight_scale, gmm2_weight, gmm2_weight_scale):
    act_col = x_active_mask.astype(jnp.float32).reshape(B, 1)
    smooth3 = smooth_scales.reshape(LOCAL_E, 1, H)
    s1_3 = gmm1_weight_scale.reshape(LOCAL_E, 1, 2 * I)
    s2_3 = gmm2_weight_scale.reshape(LOCAL_E, 1, H)

    grid = (LOCAL_E, NI)
    out = pl.pallas_call(
        _ffn_body,
        grid=grid,
        in_specs=[
            pl.BlockSpec((B, K), lambda e, i: (0, 0)),
            pl.BlockSpec((B, K), lambda e, i: (0, 0)),
            pl.BlockSpec((B, 1), lambda e, i: (0, 0)),
            pl.BlockSpec((B, H), lambda e, i: (0, 0)),
            pl.BlockSpec((1, 1, H), lambda e, i: (e, 0, 0)),
            pl.BlockSpec((1, HH, IT), lambda e, i: (e, 0, i)),
            pl.BlockSpec((1, HH, IT), lambda e, i: (e, 1, i)),
            pl.BlockSpec((1, HH, IT), lambda e, i: (e, 0, NI + i)),
            pl.BlockSpec((1, HH, IT), lambda e, i: (e, 1, NI + i)),
            pl.BlockSpec((1, 1, IT), lambda e, i: (e, 0, i)),
            pl.BlockSpec((1, 1, IT), lambda e, i: (e, 0, NI + i)),
            pl.BlockSpec((1, IT, HH), lambda e, i: (e, i, 0)),
            pl.BlockSpec((1, IT, HH), lambda e, i: (e, i, 1)),
            pl.BlockSpec((1, 1, H), lambda e, i: (e, 0, 0)),
        ],
        out_specs=pl.BlockSpec((B, H), lambda e, i: (0, 0)),
        out_shape=jax.ShapeDtypeStruct((B, H), jnp.float32),
        compiler_params=pltpu.CompilerParams(
            dimension_semantics=("arbitrary", "arbitrary"),
        ),
    )(expert_ids, expert_scales, act_col, x, smooth3,
      gmm1_weight, gmm1_weight, gmm1_weight, gmm1_weight,
      s1_3, s1_3,
      gmm2_weight, gmm2_weight, s2_3)
    return out


# speedup vs baseline: 1.0731x; 1.0027x over previous
import jax
import jax.numpy as jnp
from jax.experimental import pallas as pl
from jax.experimental.pallas import tpu as pltpu

B = 128
K = 8
LOCAL_E = 8
H = 2048
HQ = H // 4
I = 1024
IT = 512
NI = I // IT


def _ffn_body(ids_ref, scl_ref, act_ref, x_ref, smooth_ref,
              w1g0_ref, w1g1_ref, w1g2_ref, w1g3_ref,
              w1u0_ref, w1u1_ref, w1u2_ref, w1u3_ref,
              s1g_ref, s1u_ref,
              w20_ref, w21_ref, w22_ref, w23_ref, s2_ref,
              out_ref):
    e = pl.program_id(0)
    i = pl.program_id(1)

    m = (ids_ref[...] == e).astype(jnp.float32)
    w_col = jnp.sum(m * scl_ref[...], axis=1, keepdims=True) * act_ref[...]

    xs = x_ref[...] * smooth_ref[0]
    w1g = (w1g0_ref, w1g1_ref, w1g2_ref, w1g3_ref)
    w1u = (w1u0_ref, w1u1_ref, w1u2_ref, w1u3_ref)
    g = s1g_ref[0] * sum(
        jnp.dot(xs[:, q * HQ:(q + 1) * HQ], w1g[q][0],
                preferred_element_type=jnp.float32) for q in range(4))
    u = s1u_ref[0] * sum(
        jnp.dot(xs[:, q * HQ:(q + 1) * HQ], w1u[q][0],
                preferred_element_type=jnp.float32) for q in range(4))
    a = (g * jax.nn.sigmoid(g)) * u

    @pl.when((e == 0) & (i == 0))
    def _():
        out_ref[...] = jnp.zeros_like(out_ref)

    s2 = s2_ref[0]
    w2 = (w20_ref, w21_ref, w22_ref, w23_ref)
    for q in range(4):
        part = jnp.dot(a, w2[q][0], preferred_element_type=jnp.float32)
        sl = slice(q * HQ, (q + 1) * HQ)
        out_ref[:, sl] += part * s2[:, sl] * w_col


def kernel(x, expert_ids, smooth_scales, expert_scales, x_active_mask,
           gmm1_weight, gmm1_weight_scale, gmm2_weight, gmm2_weight_scale):
    act_col = x_active_mask.astype(jnp.float32).reshape(B, 1)
    smooth3 = smooth_scales.reshape(LOCAL_E, 1, H)
    s1_3 = gmm1_weight_scale.reshape(LOCAL_E, 1, 2 * I)
    s2_3 = gmm2_weight_scale.reshape(LOCAL_E, 1, H)

    grid = (LOCAL_E, NI)

    def w1g_spec(q):
        return pl.BlockSpec((1, HQ, IT), lambda e, i, q=q: (e, q, i))

    def w1u_spec(q):
        return pl.BlockSpec((1, HQ, IT), lambda e, i, q=q: (e, q, NI + i))

    def w2_spec(q):
        return pl.BlockSpec((1, IT, HQ), lambda e, i, q=q: (e, i, q))

    out = pl.pallas_call(
        _ffn_body,
        grid=grid,
        in_specs=[
            pl.BlockSpec((B, K), lambda e, i: (0, 0)),
            pl.BlockSpec((B, K), lambda e, i: (0, 0)),
            pl.BlockSpec((B, 1), lambda e, i: (0, 0)),
            pl.BlockSpec((B, H), lambda e, i: (0, 0)),
            pl.BlockSpec((1, 1, H), lambda e, i: (e, 0, 0)),
            w1g_spec(0), w1g_spec(1), w1g_spec(2), w1g_spec(3),
            w1u_spec(0), w1u_spec(1), w1u_spec(2), w1u_spec(3),
            pl.BlockSpec((1, 1, IT), lambda e, i: (e, 0, i)),
            pl.BlockSpec((1, 1, IT), lambda e, i: (e, 0, NI + i)),
            w2_spec(0), w2_spec(1), w2_spec(2), w2_spec(3),
            pl.BlockSpec((1, 1, H), lambda e, i: (e, 0, 0)),
        ],
        out_specs=pl.BlockSpec((B, H), lambda e, i: (0, 0)),
        out_shape=jax.ShapeDtypeStruct((B, H), jnp.float32),
        compiler_params=pltpu.CompilerParams(
            dimension_semantics=("arbitrary", "arbitrary"),
        ),
    )(expert_ids, expert_scales, act_col, x, smooth3,
      gmm1_weight, gmm1_weight, gmm1_weight, gmm1_weight,
      gmm1_weight, gmm1_weight, gmm1_weight, gmm1_weight,
      s1_3, s1_3,
      gmm2_weight, gmm2_weight, gmm2_weight, gmm2_weight, s2_3)
    return out
